# baseline (speedup 1.0000x reference)
import functools

import jax
import jax.numpy as jnp
from jax.experimental import pallas as pl


def _abig_kernel(ei_ref, abig_ref, *, G, J, Eper, H):
    R = G * J
    RP = -(-R // 128) * 128
    f32 = jnp.float32
    row_node = jax.lax.broadcasted_iota(jnp.int32, (R, Eper), 0) % J
    src = ei_ref[0:1, :]
    dst = ei_ref[1:2, :]
    src_oh = (row_node == src).astype(f32)
    dst_oh = (row_node == dst).astype(f32)
    tiledA = jax.lax.dot_general(
        dst_oh, src_oh, (((1,), (1,)), ((), ())),
        preferred_element_type=f32)
    ri = jax.lax.broadcasted_iota(jnp.int32, (R, R), 0)
    ci = jax.lax.broadcasted_iota(jnp.int32, (R, R), 1)
    same_graph = (ri // J) == (ci // J)
    Abig = jnp.where(same_graph, tiledA, 0.0) + (ri == ci).astype(f32)
    Apad = jnp.pad(Abig, ((0, 0), (0, RP - R)))
    abig_ref[...] = jnp.concatenate([Apad] * H, axis=1)


def _gat_kernel(x_ref, abig_ref, w_ref, acat_ref, bias_ref, out_ref, *, G, J, H, C):
    D = H * C
    R = G * J
    RP = -(-R // 128) * 128
    f32 = jnp.float32

    x2 = x_ref[...].reshape(R, D)
    h = jnp.dot(x2, w_ref[...], preferred_element_type=f32)
    a = jnp.dot(h, acat_ref[...], preferred_element_type=f32)
    ap = jnp.pad(a, ((0, RP - R), (0, 0)))

    ones = jnp.ones((R, 1), dtype=f32)
    onesp = jnp.ones((RP, 1), dtype=f32)
    outs = []
    for hh in range(H):
        a_dst_h = a[:, H + hh:H + hh + 1]
        a_src_h = ap[:, hh:hh + 1]
        lhs = jnp.concatenate([a_dst_h, ones], axis=1)
        rhs = jnp.concatenate([onesp, a_src_h], axis=1)
        alpha = jax.lax.dot_general(
            lhs, rhs, (((1,), (1,)), ((), ())),
            preferred_element_type=f32)
        alpha = jnp.maximum(alpha, 0.2 * alpha)
        w = jnp.exp(alpha) * abig_ref[:, RP * hh:RP * hh + RP]
        denom = jnp.sum(w, axis=1, keepdims=True)
        out_h = jax.lax.dot_general(
            w[:, :R], h[:, hh * C:(hh + 1) * C],
            (((1,), (0,)), ((), ())),
            preferred_element_type=f32)
        outs.append(out_h / (denom + 1e-16))
    out = jnp.concatenate(outs, axis=-1)
    out = out + bias_ref[...]
    out_ref[...] = out.reshape(G, J, D)


@jax.jit
def kernel(x, edge_index, W, att_src, att_dst, bias):
    b, t, j, d = x.shape
    BT = b * t
    H = att_src.shape[1]
    C = att_src.shape[2]
    Eper = edge_index.shape[1]
    G = 40
    R = G * j
    RP = -(-R // 128) * 128

    x3 = x.reshape(BT, j, d)
    eyeH = jnp.eye(H, dtype=jnp.float32)
    asrc_mat = (att_src.reshape(H, C)[:, :, None] * eyeH[:, None, :]).reshape(d, H)
    adst_mat = (att_dst.reshape(H, C)[:, :, None] * eyeH[:, None, :]).reshape(d, H)
    acat = jnp.concatenate([asrc_mat, adst_mat], axis=1)
    bias2 = bias.reshape(1, d)

    abig = pl.pallas_call(
        functools.partial(_abig_kernel, G=G, J=j, Eper=Eper, H=H),
        grid=(1,),
        in_specs=[pl.BlockSpec((2, Eper), lambda i: (0, 0))],
        out_specs=pl.BlockSpec((R, RP * H), lambda i: (0, 0)),
        out_shape=jax.ShapeDtypeStruct((R, RP * H), jnp.float32),
    )(edge_index)

    grid = (BT // G,)
    out = pl.pallas_call(
        functools.partial(_gat_kernel, G=G, J=j, H=H, C=C),
        grid=grid,
        in_specs=[
            pl.BlockSpec((G, j, d), lambda i: (i, 0, 0)),
            pl.BlockSpec((R, RP * H), lambda i: (0, 0)),
            pl.BlockSpec((d, d), lambda i: (0, 0)),
            pl.BlockSpec((d, 2 * H), lambda i: (0, 0)),
            pl.BlockSpec((1, d), lambda i: (0, 0)),
        ],
        out_specs=pl.BlockSpec((G, j, d), lambda i: (i, 0, 0)),
        out_shape=jax.ShapeDtypeStruct((BT, j, d), jnp.float32),
    )(x3, abig, W, acat, bias2)
    return out.reshape(b, t, j, d)


# speedup vs baseline: 1.0690x; 1.0690x over previous
import functools

import jax
import jax.numpy as jnp
from jax.experimental import pallas as pl

SB = 8
NS = 4


def _abig_kernel(ei_ref, abig_ref, *, J, Eper, H):
    R = SB * J
    RP = -(-R // 128) * 128
    f32 = jnp.float32
    row_node = jax.lax.broadcasted_iota(jnp.int32, (R, Eper), 0) % J
    src = ei_ref[0:1, :]
    dst = ei_ref[1:2, :]
    src_oh = (row_node == src).astype(f32)
    dst_oh = (row_node == dst).astype(f32)
    tiledA = jax.lax.dot_general(
        dst_oh, src_oh, (((1,), (1,)), ((), ())),
        preferred_element_type=f32)
    ri = jax.lax.broadcasted_iota(jnp.int32, (R, R), 0)
    ci = jax.lax.broadcasted_iota(jnp.int32, (R, R), 1)
    same_graph = (ri // J) == (ci // J)
    Abig = jnp.where(same_graph, tiledA, 0.0) + (ri == ci).astype(f32)
    Apad = jnp.pad(Abig, ((0, 0), (0, RP - R)))
    abig_ref[...] = jnp.concatenate([Apad] * H, axis=1)


def _gat_kernel(x_ref, abig_ref, w_ref, acat_ref, bias_ref, out_ref, *, J, H, C):
    D = H * C
    R = SB * J
    RP = -(-R // 128) * 128
    G = SB * NS
    f32 = jnp.float32

    x2 = x_ref[...].reshape(G * J, D)
    h = jnp.dot(x2, w_ref[...], preferred_element_type=f32)
    a = jnp.dot(h, acat_ref[...], preferred_element_type=f32)

    ones = jnp.ones((R, 1), dtype=f32)
    onesp = jnp.ones((RP, 1), dtype=f32)
    sub_outs = []
    for kk in range(NS):
        h_sb = h[kk * R:(kk + 1) * R, :]
        a_sb = a[kk * R:(kk + 1) * R, :]
        ap = jnp.pad(a_sb, ((0, RP - R), (0, 0)))
        outs = []
        for hh in range(H):
            a_dst_h = a_sb[:, H + hh:H + hh + 1]
            a_src_h = ap[:, hh:hh + 1]
            lhs = jnp.concatenate([a_dst_h, ones], axis=1)
            rhs = jnp.concatenate([onesp, a_src_h], axis=1)
            alpha = jax.lax.dot_general(
                lhs, rhs, (((1,), (1,)), ((), ())),
                preferred_element_type=f32)
            alpha = jnp.maximum(alpha, 0.2 * alpha)
            w = jnp.exp(alpha) * abig_ref[:, RP * hh:RP * hh + RP]
            denom = jnp.sum(w, axis=1, keepdims=True)
            out_h = jax.lax.dot_general(
                w[:, :R], h_sb[:, hh * C:(hh + 1) * C],
                (((1,), (0,)), ((), ())),
                preferred_element_type=f32)
            outs.append(out_h / (denom + 1e-16))
        sub_outs.append(jnp.concatenate(outs, axis=-1))
    out = jnp.concatenate(sub_outs, axis=0)
    out = out + bias_ref[...]
    out_ref[...] = out.reshape(G, J, D)


@jax.jit
def kernel(x, edge_index, W, att_src, att_dst, bias):
    b, t, j, d = x.shape
    BT = b * t
    H = att_src.shape[1]
    C = att_src.shape[2]
    Eper = edge_index.shape[1]
    G = SB * NS
    R = SB * j
    RP = -(-R // 128) * 128

    x3 = x.reshape(BT, j, d)
    eyeH = jnp.eye(H, dtype=jnp.float32)
    asrc_mat = (att_src.reshape(H, C)[:, :, None] * eyeH[:, None, :]).reshape(d, H)
    adst_mat = (att_dst.reshape(H, C)[:, :, None] * eyeH[:, None, :]).reshape(d, H)
    acat = jnp.concatenate([asrc_mat, adst_mat], axis=1)
    bias2 = bias.reshape(1, d)

    abig = pl.pallas_call(
        functools.partial(_abig_kernel, J=j, Eper=Eper, H=H),
        grid=(1,),
        in_specs=[pl.BlockSpec((2, Eper), lambda i: (0, 0))],
        out_specs=pl.BlockSpec((R, RP * H), lambda i: (0, 0)),
        out_shape=jax.ShapeDtypeStruct((R, RP * H), jnp.float32),
    )(edge_index)

    grid = (BT // G,)
    out = pl.pallas_call(
        functools.partial(_gat_kernel, J=j, H=H, C=C),
        grid=grid,
        in_specs=[
            pl.BlockSpec((G, j, d), lambda i: (i, 0, 0)),
            pl.BlockSpec((R, RP * H), lambda i: (0, 0)),
            pl.BlockSpec((d, d), lambda i: (0, 0)),
            pl.BlockSpec((d, 2 * H), lambda i: (0, 0)),
            pl.BlockSpec((1, d), lambda i: (0, 0)),
        ],
        out_specs=pl.BlockSpec((G, j, d), lambda i: (i, 0, 0)),
        out_shape=jax.ShapeDtypeStruct((BT, j, d), jnp.float32),
    )(x3, abig, W, acat, bias2)
    return out.reshape(b, t, j, d)


# speedup vs baseline: 1.2215x; 1.1426x over previous
import functools

import jax
import jax.numpy as jnp
from jax.experimental import pallas as pl


def _abig_kernel(ei_ref, abig_ref, *, G, J, Eper, H):
    R = G * J
    RP = -(-R // 128) * 128
    f32 = jnp.float32
    row_node = jax.lax.broadcasted_iota(jnp.int32, (R, Eper), 0) % J
    src = ei_ref[0:1, :]
    dst = ei_ref[1:2, :]
    src_oh = (row_node == src).astype(f32)
    dst_oh = (row_node == dst).astype(f32)
    tiledA = jax.lax.dot_general(
        dst_oh, src_oh, (((1,), (1,)), ((), ())),
        preferred_element_type=f32)
    ri = jax.lax.broadcasted_iota(jnp.int32, (R, R), 0)
    ci = jax.lax.broadcasted_iota(jnp.int32, (R, R), 1)
    same_graph = (ri // J) == (ci // J)
    Abig = jnp.where(same_graph, tiledA, 0.0) + (ri == ci).astype(f32)
    Apad = jnp.pad(Abig, ((0, 0), (0, RP - R)))
    abig_ref[...] = jnp.concatenate([Apad] * H, axis=1)


def _gat_kernel(x_ref, abig_ref, w_ref, acat_ref, bias_ref, out_ref, *, G, J, H, C):
    D = H * C
    R = G * J
    RP = -(-R // 128) * 128
    f32 = jnp.float32

    x2 = x_ref[...].reshape(R, D)
    h = jnp.dot(x2, w_ref[...], preferred_element_type=f32)
    a = jnp.dot(h, acat_ref[...], preferred_element_type=f32)
    e1 = jnp.exp(a)
    e2 = jnp.exp(0.2 * a)
    e1p = jnp.pad(e1, ((0, RP - R), (0, 0)))
    e2p = jnp.pad(e2, ((0, RP - R), (0, 0)))

    ones = jnp.ones((R, 1), dtype=f32)
    outs = []
    for hh in range(H):
        w1 = jax.lax.dot_general(
            e1[:, H + hh:H + hh + 1], e1p[:, hh:hh + 1],
            (((1,), (1,)), ((), ())),
            preferred_element_type=f32)
        w2 = jax.lax.dot_general(
            e2[:, H + hh:H + hh + 1], e2p[:, hh:hh + 1],
            (((1,), (1,)), ((), ())),
            preferred_element_type=f32)
        w = jnp.maximum(w1, w2) * abig_ref[:, RP * hh:RP * hh + RP]
        rhs_aug = jnp.concatenate(
            [h[:, hh * C:(hh + 1) * C], ones], axis=1)
        out_aug = jax.lax.dot_general(
            w[:, :R], rhs_aug, (((1,), (0,)), ((), ())),
            preferred_element_type=f32)
        outs.append(out_aug[:, :C] / (out_aug[:, C:C + 1] + 1e-16))
    out = jnp.concatenate(outs, axis=-1)
    out = out + bias_ref[...]
    out_ref[...] = out.reshape(G, J, D)


@jax.jit
def kernel(x, edge_index, W, att_src, att_dst, bias):
    b, t, j, d = x.shape
    BT = b * t
    H = att_src.shape[1]
    C = att_src.shape[2]
    Eper = edge_index.shape[1]
    G = 20
    R = G * j
    RP = -(-R // 128) * 128

    x3 = x.reshape(BT, j, d)
    eyeH = jnp.eye(H, dtype=jnp.float32)
    asrc_mat = (att_src.reshape(H, C)[:, :, None] * eyeH[:, None, :]).reshape(d, H)
    adst_mat = (att_dst.reshape(H, C)[:, :, None] * eyeH[:, None, :]).reshape(d, H)
    acat = jnp.concatenate([asrc_mat, adst_mat], axis=1)
    bias2 = bias.reshape(1, d)

    abig = pl.pallas_call(
        functools.partial(_abig_kernel, G=G, J=j, Eper=Eper, H=H),
        grid=(1,),
        in_specs=[pl.BlockSpec((2, Eper), lambda i: (0, 0))],
        out_specs=pl.BlockSpec((R, RP * H), lambda i: (0, 0)),
        out_shape=jax.ShapeDtypeStruct((R, RP * H), jnp.float32),
    )(edge_index)

    grid = (BT // G,)
    out = pl.pallas_call(
        functools.partial(_gat_kernel, G=G, J=j, H=H, C=C),
        grid=grid,
        in_specs=[
            pl.BlockSpec((G, j, d), lambda i: (i, 0, 0)),
            pl.BlockSpec((R, RP * H), lambda i: (0, 0)),
            pl.BlockSpec((d, d), lambda i: (0, 0)),
            pl.BlockSpec((d, 2 * H), lambda i: (0, 0)),
            pl.BlockSpec((1, d), lambda i: (0, 0)),
        ],
        out_specs=pl.BlockSpec((G, j, d), lambda i: (i, 0, 0)),
        out_shape=jax.ShapeDtypeStruct((BT, j, d), jnp.float32),
    )(x3, abig, W, acat, bias2)
    return out.reshape(b, t, j, d)


# speedup vs baseline: 1.2710x; 1.0406x over previous
import functools

import jax
import jax.numpy as jnp
from jax.experimental import pallas as pl


def _abig_kernel(ei_ref, abig_ref, *, G, J, Eper, H):
    R = G * J
    RP = -(-R // 128) * 128
    f32 = jnp.float32
    row_node = jax.lax.broadcasted_iota(jnp.int32, (R, Eper), 0) % J
    src = ei_ref[0:1, :]
    dst = ei_ref[1:2, :]
    src_oh = (row_node == src).astype(f32)
    dst_oh = (row_node == dst).astype(f32)
    tiledA = jax.lax.dot_general(
        dst_oh, src_oh, (((1,), (1,)), ((), ())),
        preferred_element_type=f32)
    ri = jax.lax.broadcasted_iota(jnp.int32, (R, R), 0)
    ci = jax.lax.broadcasted_iota(jnp.int32, (R, R), 1)
    same_graph = (ri // J) == (ci // J)
    Abig = jnp.where(same_graph, tiledA, 0.0) + (ri == ci).astype(f32)
    Apad = jnp.pad(Abig, ((0, 0), (0, RP - R)))
    abig_ref[...] = jnp.concatenate([Apad] * H, axis=1)


def _gat_kernel(x_ref, abig_ref, w_ref, acat_ref, bias_ref, out_ref, *, G, J, H, C):
    D = H * C
    R = G * J
    RP = -(-R // 128) * 128
    f32 = jnp.float32

    bf16 = jnp.bfloat16
    x2 = x_ref[...].reshape(R, D).astype(bf16)
    h = jnp.dot(x2, w_ref[...], preferred_element_type=f32)
    a = jnp.dot(h.astype(bf16), acat_ref[...],
                preferred_element_type=f32)
    ap = jnp.pad(a, ((0, RP - R), (0, 0)))
    hb = h.astype(bf16)

    ones = jnp.ones((R, 1), dtype=f32)
    onesp = jnp.ones((RP, 1), dtype=f32)
    outs = []
    for hh in range(H):
        a_dst_h = a[:, H + hh:H + hh + 1]
        a_src_h = ap[:, hh:hh + 1]
        lhs = jnp.concatenate([a_dst_h, ones], axis=1)
        rhs = jnp.concatenate([onesp, a_src_h], axis=1)
        alpha = jax.lax.dot_general(
            lhs, rhs, (((1,), (1,)), ((), ())),
            preferred_element_type=f32)
        alpha = jnp.maximum(alpha, 0.2 * alpha)
        w = jnp.exp(alpha) * abig_ref[:, RP * hh:RP * hh + RP]
        rhs_aug = jnp.concatenate(
            [hb[:, hh * C:(hh + 1) * C], jnp.ones((R, 1), dtype=bf16)],
            axis=1)
        out_aug = jax.lax.dot_general(
            w[:, :R].astype(bf16), rhs_aug, (((1,), (0,)), ((), ())),
            preferred_element_type=f32)
        outs.append(out_aug[:, :C] / (out_aug[:, C:C + 1] + 1e-16))
    out = jnp.concatenate(outs, axis=-1)
    out = out + bias_ref[...]
    out_ref[...] = out.reshape(G, J, D)


@jax.jit
def kernel(x, edge_index, W, att_src, att_dst, bias):
    b, t, j, d = x.shape
    BT = b * t
    H = att_src.shape[1]
    C = att_src.shape[2]
    Eper = edge_index.shape[1]
    G = 20
    R = G * j
    RP = -(-R // 128) * 128

    x3 = x.reshape(BT, j, d)
    eyeH = jnp.eye(H, dtype=jnp.float32)
    asrc_mat = (att_src.reshape(H, C)[:, :, None] * eyeH[:, None, :]).reshape(d, H)
    adst_mat = (att_dst.reshape(H, C)[:, :, None] * eyeH[:, None, :]).reshape(d, H)
    acat = jnp.concatenate([asrc_mat, adst_mat], axis=1)
    bias2 = bias.reshape(1, d)

    abig = pl.pallas_call(
        functools.partial(_abig_kernel, G=G, J=j, Eper=Eper, H=H),
        grid=(1,),
        in_specs=[pl.BlockSpec((2, Eper), lambda i: (0, 0))],
        out_specs=pl.BlockSpec((R, RP * H), lambda i: (0, 0)),
        out_shape=jax.ShapeDtypeStruct((R, RP * H), jnp.float32),
    )(edge_index)

    grid = (BT // G,)
    out = pl.pallas_call(
        functools.partial(_gat_kernel, G=G, J=j, H=H, C=C),
        grid=grid,
        in_specs=[
            pl.BlockSpec((G, j, d), lambda i: (i, 0, 0)),
            pl.BlockSpec((R, RP * H), lambda i: (0, 0)),
            pl.BlockSpec((d, d), lambda i: (0, 0)),
            pl.BlockSpec((d, 2 * H), lambda i: (0, 0)),
            pl.BlockSpec((1, d), lambda i: (0, 0)),
        ],
        out_specs=pl.BlockSpec((G, j, d), lambda i: (i, 0, 0)),
        out_shape=jax.ShapeDtypeStruct((BT, j, d), jnp.float32),
    )(x3, abig, W.astype(jnp.bfloat16), acat.astype(jnp.bfloat16), bias2)
    return out.reshape(b, t, j, d)
